# baseline (speedup 1.0000x reference)
import functools

import jax
import jax.numpy as jnp
from jax import lax
from jax.experimental import pallas as pl
from jax.experimental.pallas import tpu as pltpu
from jax.experimental.pallas import tpu_sc as plsc

_N = 10000
_E = 320000
_DIN = 128
_H = 64
_B = 64
_NUM_STEPS = 2
_S2S_STEPS = 3

_CHUNK = 128
_NCHUNKS = _E // _CHUNK
_NW = 32
_CPW = 80
_NCHUNKS_PAD = _CPW * _NW
_TPC = 16
_RPT = 632
_NPAD = _RPT * _TPC

_HIGH = lax.Precision.HIGHEST


def _mm(a, b):
    return jnp.dot(a, b, preferred_element_type=jnp.float32, precision=_HIGH)


_NBUF = 5
_NGRP = _CPW // _NBUF


def _seg_sum_body(table, src2, dst2, zeros, out, src_v, dst_v, rows,
                  acc, gsem, ssem):
    c = lax.axis_index("c")
    s = lax.axis_index("s")
    w = s * 2 + c

    pltpu.sync_copy(zeros.at[pl.ds(s * _RPT, _RPT)],
                    acc.at[pl.ds(s * _RPT, _RPT)])
    base = w * _CPW
    pltpu.sync_copy(src2.at[pl.ds(base, _CPW)], src_v)
    pltpu.sync_copy(dst2.at[pl.ds(base, _CPW)], dst_v)
    plsc.subcore_barrier()

    for b in range(_NBUF):
        pltpu.async_copy(table.at[src_v.at[b]], rows.at[b], gsem)

    def group(g, carry):
        gbase = g * _NBUF
        for b in range(_NBUF):
            pltpu.make_async_copy(table.at[src_v.at[gbase + b]],
                                  rows.at[b], gsem).wait()
            pltpu.async_copy(rows.at[b], acc.at[dst_v.at[gbase + b]], ssem,
                             add=True)
        for b in range(_NBUF):
            pltpu.make_async_copy(rows.at[b], acc.at[dst_v.at[gbase + b]],
                                  ssem).wait()

        @pl.when(g + 1 < _NGRP)
        def _():
            for b in range(_NBUF):
                pltpu.async_copy(table.at[src_v.at[gbase + _NBUF + b]],
                                 rows.at[b], gsem)

        return carry

    lax.fori_loop(0, _NGRP, group, 0)

    plsc.subcore_barrier()
    pltpu.sync_copy(acc.at[pl.ds(s * _RPT, _RPT)],
                    out.at[c, pl.ds(s * _RPT, _RPT)])


@functools.cache
def _get_seg_sum_sc():
    mesh = plsc.VectorSubcoreMesh(core_axis_name="c", subcore_axis_name="s")
    return pl.kernel(
        _seg_sum_body,
        mesh=mesh,
        out_type=jax.ShapeDtypeStruct((2, _NPAD, _H), jnp.float32),
        scratch_types=[
            pltpu.VMEM((_CPW, _CHUNK), jnp.int32),
            pltpu.VMEM((_CPW, _CHUNK), jnp.int32),
            pltpu.VMEM((_NBUF, _CHUNK, _H), jnp.float32),
            pltpu.VMEM_SHARED((_NPAD, _H), jnp.float32),
            pltpu.SemaphoreType.DMA,
            pltpu.SemaphoreType.DMA,
        ],
        compiler_params=pltpu.CompilerParams(use_tc_tiling_on_sc=False),
    )


def _mlp_body(x_ref, w_ref, b_ref, o_ref):
    o_ref[...] = _mm(x_ref[...], w_ref[...]) + b_ref[...]


_BLK = 1000
_NBLK = _N // _BLK


def _row_spec(cols):
    return pl.BlockSpec((_BLK, cols), lambda i: (i, 0))


def _full_spec(shape):
    return pl.BlockSpec(shape, lambda i: (0, 0))


_mlp_call = pl.pallas_call(
    _mlp_body,
    grid=(_NBLK,),
    in_specs=[_row_spec(_DIN), _full_spec((_DIN, _H)), _full_spec((1, _H))],
    out_specs=_row_spec(_H),
    out_shape=jax.ShapeDtypeStruct((_N, _H), jnp.float32))


def _gru_body(p0, p1, x, h, wconv, bconv, wmr, wmz, wmn, wxr, wxz, wxn,
              ur, uz, un, br, bz, bin_, bhn, o):
    agg = p0[0] + p1[0]
    m = jax.nn.relu(_mm(agg, wconv[...]) + bconv[...])
    xv = x[...]
    hv = h[...]
    r = jax.nn.sigmoid(_mm(m, wmr[...]) + _mm(xv, wxr[...])
                       + _mm(hv, ur[...]) + br[...])
    z = jax.nn.sigmoid(_mm(m, wmz[...]) + _mm(xv, wxz[...])
                       + _mm(hv, uz[...]) + bz[...])
    hn = _mm(hv, un[...]) + bhn[...]
    nn = jnp.tanh(_mm(m, wmn[...]) + _mm(xv, wxn[...]) + bin_[...] + r * hn)
    o[...] = (1.0 - z) * nn + z * hv


_gru_call = pl.pallas_call(
    _gru_body,
    grid=(_NBLK,),
    in_specs=[pl.BlockSpec((1, _BLK, _H), lambda i: (0, i, 0)),
              pl.BlockSpec((1, _BLK, _H), lambda i: (1, i, 0)),
              _row_spec(_DIN), _row_spec(_H),
              _full_spec((_H, _H)), _full_spec((1, _H)),
              _full_spec((_H, _H)), _full_spec((_H, _H)), _full_spec((_H, _H)),
              _full_spec((_DIN, _H)), _full_spec((_DIN, _H)),
              _full_spec((_DIN, _H)),
              _full_spec((_H, _H)), _full_spec((_H, _H)), _full_spec((_H, _H)),
              _full_spec((1, _H)), _full_spec((1, _H)), _full_spec((1, _H)),
              _full_spec((1, _H))],
    out_specs=_row_spec(_H),
    out_shape=jax.ShapeDtypeStruct((_N, _H), jnp.float32))


def _s2s_body(out_ref, bat_ref, wq, wr, u, ball, w1q, w1r, b1r, w2, b2r,
              o_ref):
    outv = out_ref[...]
    ids = bat_ref[...]
    cols = lax.broadcasted_iota(jnp.int32, (_N, _B), 1)
    mb = ids == cols
    mf = jnp.where(mb, 1.0, 0.0).astype(jnp.float32)

    q = jnp.zeros((_B, _H), jnp.float32)
    rv = jnp.zeros((_B, _H), jnp.float32)
    hs = jnp.zeros((_B, _H), jnp.float32)
    cs = jnp.zeros((_B, _H), jnp.float32)
    for _ in range(_S2S_STEPS):
        g = _mm(q, wq[...]) + _mm(rv, wr[...]) + _mm(hs, u[...]) + ball[...]
        ig = jax.nn.sigmoid(g[:, :_H])
        fg = jax.nn.sigmoid(g[:, _H:2 * _H])
        gg = jnp.tanh(g[:, 2 * _H:3 * _H])
        og = jax.nn.sigmoid(g[:, 3 * _H:])
        cs = fg * cs + ig * gg
        hs = og * jnp.tanh(cs)
        q = hs
        qb = _mm(mf, q)
        e = jnp.sum(outv * qb, axis=1, keepdims=True)
        emax = jnp.max(jnp.where(mb, e, -1e30), axis=0, keepdims=True)
        emaxb = jnp.sum(mf * emax, axis=1, keepdims=True)
        a = jnp.exp(e - emaxb)
        den = lax.dot_general(a, mf, (((0,), (0,)), ((), ())),
                              precision=_HIGH,
                              preferred_element_type=jnp.float32)
        denb = jnp.sum(mf * den, axis=1, keepdims=True)
        a = a / (denb + 1e-16)
        rv = lax.dot_general(mf, a * outv, (((0,), (0,)), ((), ())),
                             precision=_HIGH,
                             preferred_element_type=jnp.float32)
    o1 = jax.nn.relu(_mm(q, w1q[...]) + _mm(rv, w1r[...]) + b1r[...])
    logits = _mm(o1, w2[...]) + b2r[...]
    ls = logits - jnp.max(logits, axis=1, keepdims=True)
    o_ref[...] = ls - jnp.log(jnp.sum(jnp.exp(ls), axis=1, keepdims=True))


_s2s_call = pl.pallas_call(
    _s2s_body, out_shape=jax.ShapeDtypeStruct((_B, 4), jnp.float32),
    compiler_params=pltpu.CompilerParams(vmem_limit_bytes=100 * 1024 * 1024))


def kernel(x, edge_index, edge_attr, batch, W_mlp, b_mlp, W_conv, b_conv,
           gru_Wih, gru_Whh, gru_bih, gru_bhh,
           lstm_Wih, lstm_Whh, lstm_bih, lstm_bhh, W1, b1, W2, b2):
    del edge_attr
    npad_e = (_NCHUNKS_PAD - _NCHUNKS) * _CHUNK
    src2 = jnp.concatenate(
        [edge_index[0].astype(jnp.int32), jnp.zeros((npad_e,), jnp.int32)]
    ).reshape(_NCHUNKS_PAD, _CHUNK)
    dst2 = jnp.concatenate(
        [edge_index[1].astype(jnp.int32),
         jnp.full((npad_e,), _N, jnp.int32)]
    ).reshape(_NCHUNKS_PAD, _CHUNK)
    zeros = jnp.zeros((_NPAD, _H), jnp.float32)

    wih_t = gru_Wih.T
    wm, wx = wih_t[:_H], wih_t[_H:]
    whh_t = gru_Whh.T
    wmr, wmz, wmn = wm[:, :_H], wm[:, _H:2 * _H], wm[:, 2 * _H:]
    wxr, wxz, wxn = wx[:, :_H], wx[:, _H:2 * _H], wx[:, 2 * _H:]
    ur, uz, un = whh_t[:, :_H], whh_t[:, _H:2 * _H], whh_t[:, 2 * _H:]
    br = (gru_bih[:_H] + gru_bhh[:_H]).reshape(1, _H)
    bz = (gru_bih[_H:2 * _H] + gru_bhh[_H:2 * _H]).reshape(1, _H)
    bin_ = gru_bih[2 * _H:].reshape(1, _H)
    bhn = gru_bhh[2 * _H:].reshape(1, _H)

    out = _mlp_call(x, W_mlp, b_mlp.reshape(1, _H))
    h = out
    seg_sum = _get_seg_sum_sc()
    for _ in range(_NUM_STEPS):
        parts = seg_sum(out, src2, dst2, zeros)
        h = _gru_call(parts, parts, x, h, W_conv,
                      b_conv.reshape(1, _H), wmr, wmz, wmn, wxr, wxz, wxn,
                      ur, uz, un, br, bz, bin_, bhn)
        out = h

    lwih_t = lstm_Wih.T
    wq_all, wr_all = lwih_t[:_H], lwih_t[_H:]
    u_all = lstm_Whh.T
    b_all = (lstm_bih + lstm_bhh).reshape(1, 4 * _H)
    return _s2s_call(out, batch.astype(jnp.int32).reshape(_N, 1),
                     wq_all, wr_all, u_all, b_all,
                     W1[:_H], W1[_H:], b1.reshape(1, _H),
                     W2, b2.reshape(1, 4))


# speedup vs baseline: 1.2487x; 1.2487x over previous
import functools

import jax
import jax.numpy as jnp
from jax import lax
from jax.experimental import pallas as pl
from jax.experimental.pallas import tpu as pltpu
from jax.experimental.pallas import tpu_sc as plsc

_N = 10000
_E = 320000
_DIN = 128
_H = 64
_B = 64
_NUM_STEPS = 2
_S2S_STEPS = 3

_CHUNK = 128
_NCHUNKS = _E // _CHUNK
_NW = 32
_CPW = 80
_NCHUNKS_PAD = _CPW * _NW
_TPC = 16
_RPT = 632
_NPAD = _RPT * _TPC

_HIGH = lax.Precision.DEFAULT


def _mm(a, b):
    return jnp.dot(a, b, preferred_element_type=jnp.float32, precision=_HIGH)


_NBUF = 5
_NGRP = _CPW // _NBUF


def _seg_sum_body(table, src2, dst2, zeros, out, src_v, dst_v, rows,
                  acc, gsem, ssem):
    c = lax.axis_index("c")
    s = lax.axis_index("s")
    w = s * 2 + c

    pltpu.sync_copy(zeros.at[pl.ds(s * _RPT, _RPT)],
                    acc.at[pl.ds(s * _RPT, _RPT)])
    base = w * _CPW
    pltpu.sync_copy(src2.at[pl.ds(base, _CPW)], src_v)
    pltpu.sync_copy(dst2.at[pl.ds(base, _CPW)], dst_v)
    plsc.subcore_barrier()

    for b in range(_NBUF):
        pltpu.async_copy(table.at[src_v.at[b]], rows.at[b], gsem)

    def group(g, carry):
        gbase = g * _NBUF
        for b in range(_NBUF):
            pltpu.make_async_copy(table.at[src_v.at[gbase + b]],
                                  rows.at[b], gsem).wait()
            pltpu.async_copy(rows.at[b], acc.at[dst_v.at[gbase + b]], ssem,
                             add=True)
        for b in range(_NBUF):
            pltpu.make_async_copy(rows.at[b], acc.at[dst_v.at[gbase + b]],
                                  ssem).wait()

        @pl.when(g + 1 < _NGRP)
        def _():
            for b in range(_NBUF):
                pltpu.async_copy(table.at[src_v.at[gbase + _NBUF + b]],
                                 rows.at[b], gsem)

        return carry

    lax.fori_loop(0, _NGRP, group, 0)

    plsc.subcore_barrier()
    pltpu.sync_copy(acc.at[pl.ds(s * _RPT, _RPT)],
                    out.at[c, pl.ds(s * _RPT, _RPT)])


@functools.cache
def _get_seg_sum_sc():
    mesh = plsc.VectorSubcoreMesh(core_axis_name="c", subcore_axis_name="s")
    return pl.kernel(
        _seg_sum_body,
        mesh=mesh,
        out_type=jax.ShapeDtypeStruct((2, _NPAD, _H), jnp.float32),
        scratch_types=[
            pltpu.VMEM((_CPW, _CHUNK), jnp.int32),
            pltpu.VMEM((_CPW, _CHUNK), jnp.int32),
            pltpu.VMEM((_NBUF, _CHUNK, _H), jnp.float32),
            pltpu.VMEM_SHARED((_NPAD, _H), jnp.float32),
            pltpu.SemaphoreType.DMA,
            pltpu.SemaphoreType.DMA,
        ],
        compiler_params=pltpu.CompilerParams(use_tc_tiling_on_sc=False),
    )


def _mlp_body(x_ref, w_ref, b_ref, o_ref):
    o_ref[...] = _mm(x_ref[...], w_ref[...]) + b_ref[...]


_BLK = 1000
_NBLK = _N // _BLK


def _row_spec(cols):
    return pl.BlockSpec((_BLK, cols), lambda i: (i, 0))


def _full_spec(shape):
    return pl.BlockSpec(shape, lambda i: (0, 0))


_mlp_call = pl.pallas_call(
    _mlp_body,
    grid=(_NBLK,),
    in_specs=[_row_spec(_DIN), _full_spec((_DIN, _H)), _full_spec((1, _H))],
    out_specs=_row_spec(_H),
    out_shape=jax.ShapeDtypeStruct((_N, _H), jnp.float32))


def _gru_body(p0, p1, x, h, wconv, bconv, wmr, wmz, wmn, wxr, wxz, wxn,
              ur, uz, un, br, bz, bin_, bhn, o):
    agg = p0[0] + p1[0]
    m = jax.nn.relu(_mm(agg, wconv[...]) + bconv[...])
    xv = x[...]
    hv = h[...]
    r = jax.nn.sigmoid(_mm(m, wmr[...]) + _mm(xv, wxr[...])
                       + _mm(hv, ur[...]) + br[...])
    z = jax.nn.sigmoid(_mm(m, wmz[...]) + _mm(xv, wxz[...])
                       + _mm(hv, uz[...]) + bz[...])
    hn = _mm(hv, un[...]) + bhn[...]
    nn = jnp.tanh(_mm(m, wmn[...]) + _mm(xv, wxn[...]) + bin_[...] + r * hn)
    o[...] = (1.0 - z) * nn + z * hv


_gru_call = pl.pallas_call(
    _gru_body,
    grid=(_NBLK,),
    in_specs=[pl.BlockSpec((1, _BLK, _H), lambda i: (0, i, 0)),
              pl.BlockSpec((1, _BLK, _H), lambda i: (1, i, 0)),
              _row_spec(_DIN), _row_spec(_H),
              _full_spec((_H, _H)), _full_spec((1, _H)),
              _full_spec((_H, _H)), _full_spec((_H, _H)), _full_spec((_H, _H)),
              _full_spec((_DIN, _H)), _full_spec((_DIN, _H)),
              _full_spec((_DIN, _H)),
              _full_spec((_H, _H)), _full_spec((_H, _H)), _full_spec((_H, _H)),
              _full_spec((1, _H)), _full_spec((1, _H)), _full_spec((1, _H)),
              _full_spec((1, _H))],
    out_specs=_row_spec(_H),
    out_shape=jax.ShapeDtypeStruct((_N, _H), jnp.float32))


def _s2s_body(out_ref, bat_ref, wq, wr, u, ball, w1q, w1r, b1r, w2, b2r,
              o_ref):
    outv = out_ref[...]
    ids = bat_ref[...]
    cols = lax.broadcasted_iota(jnp.int32, (_N, _B), 1)
    mb = ids == cols
    mf = jnp.where(mb, 1.0, 0.0).astype(jnp.float32)

    q = jnp.zeros((_B, _H), jnp.float32)
    rv = jnp.zeros((_B, _H), jnp.float32)
    hs = jnp.zeros((_B, _H), jnp.float32)
    cs = jnp.zeros((_B, _H), jnp.float32)
    for _ in range(_S2S_STEPS):
        g = _mm(q, wq[...]) + _mm(rv, wr[...]) + _mm(hs, u[...]) + ball[...]
        ig = jax.nn.sigmoid(g[:, :_H])
        fg = jax.nn.sigmoid(g[:, _H:2 * _H])
        gg = jnp.tanh(g[:, 2 * _H:3 * _H])
        og = jax.nn.sigmoid(g[:, 3 * _H:])
        cs = fg * cs + ig * gg
        hs = og * jnp.tanh(cs)
        q = hs
        qb = _mm(mf, q)
        e = jnp.sum(outv * qb, axis=1, keepdims=True)
        emax = jnp.max(jnp.where(mb, e, -1e30), axis=0, keepdims=True)
        emaxb = jnp.sum(mf * emax, axis=1, keepdims=True)
        a = jnp.exp(e - emaxb)
        den = lax.dot_general(a, mf, (((0,), (0,)), ((), ())),
                              precision=_HIGH,
                              preferred_element_type=jnp.float32)
        denb = jnp.sum(mf * den, axis=1, keepdims=True)
        a = a / (denb + 1e-16)
        rv = lax.dot_general(mf, a * outv, (((0,), (0,)), ((), ())),
                             precision=_HIGH,
                             preferred_element_type=jnp.float32)
    o1 = jax.nn.relu(_mm(q, w1q[...]) + _mm(rv, w1r[...]) + b1r[...])
    logits = _mm(o1, w2[...]) + b2r[...]
    ls = logits - jnp.max(logits, axis=1, keepdims=True)
    o_ref[...] = ls - jnp.log(jnp.sum(jnp.exp(ls), axis=1, keepdims=True))


_s2s_call = pl.pallas_call(
    _s2s_body, out_shape=jax.ShapeDtypeStruct((_B, 4), jnp.float32),
    compiler_params=pltpu.CompilerParams(vmem_limit_bytes=100 * 1024 * 1024))


def kernel(x, edge_index, edge_attr, batch, W_mlp, b_mlp, W_conv, b_conv,
           gru_Wih, gru_Whh, gru_bih, gru_bhh,
           lstm_Wih, lstm_Whh, lstm_bih, lstm_bhh, W1, b1, W2, b2):
    del edge_attr
    npad_e = (_NCHUNKS_PAD - _NCHUNKS) * _CHUNK
    src2 = jnp.concatenate(
        [edge_index[0].astype(jnp.int32), jnp.zeros((npad_e,), jnp.int32)]
    ).reshape(_NCHUNKS_PAD, _CHUNK)
    dst2 = jnp.concatenate(
        [edge_index[1].astype(jnp.int32),
         jnp.full((npad_e,), _N, jnp.int32)]
    ).reshape(_NCHUNKS_PAD, _CHUNK)
    zeros = jnp.zeros((_NPAD, _H), jnp.float32)

    wih_t = gru_Wih.T
    wm, wx = wih_t[:_H], wih_t[_H:]
    whh_t = gru_Whh.T
    wmr, wmz, wmn = wm[:, :_H], wm[:, _H:2 * _H], wm[:, 2 * _H:]
    wxr, wxz, wxn = wx[:, :_H], wx[:, _H:2 * _H], wx[:, 2 * _H:]
    ur, uz, un = whh_t[:, :_H], whh_t[:, _H:2 * _H], whh_t[:, 2 * _H:]
    br = (gru_bih[:_H] + gru_bhh[:_H]).reshape(1, _H)
    bz = (gru_bih[_H:2 * _H] + gru_bhh[_H:2 * _H]).reshape(1, _H)
    bin_ = gru_bih[2 * _H:].reshape(1, _H)
    bhn = gru_bhh[2 * _H:].reshape(1, _H)

    out = _mlp_call(x, W_mlp, b_mlp.reshape(1, _H))
    h = out
    seg_sum = _get_seg_sum_sc()
    for _ in range(_NUM_STEPS):
        parts = seg_sum(out, src2, dst2, zeros)
        h = _gru_call(parts, parts, x, h, W_conv,
                      b_conv.reshape(1, _H), wmr, wmz, wmn, wxr, wxz, wxn,
                      ur, uz, un, br, bz, bin_, bhn)
        out = h

    lwih_t = lstm_Wih.T
    wq_all, wr_all = lwih_t[:_H], lwih_t[_H:]
    u_all = lstm_Whh.T
    b_all = (lstm_bih + lstm_bhh).reshape(1, 4 * _H)
    return _s2s_call(out, batch.astype(jnp.int32).reshape(_N, 1),
                     wq_all, wr_all, u_all, b_all,
                     W1[:_H], W1[_H:], b1.reshape(1, _H),
                     W2, b2.reshape(1, 4))
